# baseline (speedup 1.0000x reference)
import functools

import jax
import jax.numpy as jnp
from jax import lax
from jax.experimental import pallas as pl
from jax.experimental.pallas import tpu as pltpu
from jax.experimental.pallas import tpu_sc as plsc

N = 10000
D = 128
E = 320000
NC = 2
NS = 16
NW = NC * NS
L = 16
NPAD = 10240
STRIPE = NPAD // NS

CHUNK = 64
EPW = 10240
EP = NW * EPW
NCHUNK = EPW // CHUNK
GRP = 8
NG = NCHUNK // GRP
NBUF = 4

_mesh = plsc.VectorSubcoreMesh(core_axis_name="c", subcore_axis_name="s")


@functools.partial(
    pl.kernel,
    out_type=jax.ShapeDtypeStruct((NC, NPAD), jnp.float32),
    mesh=_mesh,
    scratch_types=[
        pltpu.VMEM((GRP, CHUNK), jnp.int32),
        pltpu.VMEM((CHUNK,), jnp.float32),
        pltpu.VMEM((STRIPE,), jnp.float32),
        pltpu.VMEM_SHARED((NPAD,), jnp.float32),
    ],
)
def _deg_kernel(dst_hbm, deg_hbm, dstv, onesv, zbuf, shacc):
    c = lax.axis_index("c")
    s = lax.axis_index("s")
    w = s * NC + c

    def _zero(i, _):
        zbuf[pl.ds(i * L, L)] = jnp.zeros((L,), jnp.float32)
        return 0

    lax.fori_loop(0, STRIPE // L, _zero, 0)

    def _ones(i, _):
        onesv[pl.ds(i * L, L)] = jnp.ones((L,), jnp.float32)
        return 0

    lax.fori_loop(0, CHUNK // L, _ones, 0)

    pltpu.sync_copy(zbuf, shacc.at[pl.ds(s * STRIPE, STRIPE)])
    plsc.subcore_barrier()

    def _group(g, _):
        pltpu.sync_copy(dst_hbm.at[w, g], dstv)

        def _body(j, _):
            pltpu.sync_copy(onesv, shacc.at[dstv.at[j]], add=True)
            return 0

        lax.fori_loop(0, GRP, _body, 0)
        return 0

    lax.fori_loop(0, NG, _group, 0)
    plsc.subcore_barrier()
    pltpu.sync_copy(shacc.at[pl.ds(s * STRIPE, STRIPE)],
                    deg_hbm.at[c, pl.ds(s * STRIPE, STRIPE)])


@functools.partial(
    pl.kernel,
    out_type=jax.ShapeDtypeStruct((NC, NPAD, D), jnp.float32),
    mesh=_mesh,
    scratch_types=[
        pltpu.VMEM((GRP, CHUNK), jnp.int32),
        pltpu.VMEM((GRP, CHUNK), jnp.int32),
        [pltpu.VMEM((CHUNK, D), jnp.float32)] * NBUF,
        [pltpu.SemaphoreType.DMA] * NBUF,
        [pltpu.SemaphoreType.DMA] * NBUF,
        pltpu.VMEM_SHARED((NPAD, D), jnp.float32),
    ],
)
def _edge_kernel(y_hbm, src_hbm, dst_hbm, acc_hbm, srcv, dstv, bufs,
                 gsems, ssems, shacc):
    c = lax.axis_index("c")
    s = lax.axis_index("s")
    w = s * NC + c

    def _zero(i, _):
        r = i // (D // L)
        q = i % (D // L)
        bufs[0][r, pl.ds(q * L, L)] = jnp.zeros((L,), jnp.float32)
        return 0

    lax.fori_loop(0, CHUNK * (D // L), _zero, 0)

    def _zcopy(k, _):
        pltpu.sync_copy(bufs[0], shacc.at[pl.ds(s * STRIPE + k * CHUNK, CHUNK)])
        return 0

    lax.fori_loop(0, STRIPE // CHUNK, _zcopy, 0)
    plsc.subcore_barrier()

    def _group(g, _):
        pltpu.sync_copy(src_hbm.at[w, g], srcv)
        pltpu.sync_copy(dst_hbm.at[w, g], dstv)
        for k in range(NBUF - 1):
            pltpu.async_copy(y_hbm.at[srcv.at[k]], bufs[k], gsems[k])
        for k in range(GRP):
            b = k % NBUF
            pltpu.make_async_copy(y_hbm.at[srcv.at[k]], bufs[b],
                                  gsems[b]).wait()
            pltpu.async_copy(bufs[b], shacc.at[dstv.at[k]], ssems[b],
                             add=True)
            if k + NBUF - 1 < GRP:
                nb = (k + NBUF - 1) % NBUF
                if k - 1 >= 0:
                    pltpu.make_async_copy(bufs[nb], shacc.at[dstv.at[k - 1]],
                                          ssems[nb]).wait()
                pltpu.async_copy(y_hbm.at[srcv.at[k + NBUF - 1]], bufs[nb],
                                 gsems[nb])
        for k in range(GRP - NBUF, GRP):
            b = k % NBUF
            pltpu.make_async_copy(bufs[b], shacc.at[dstv.at[k]],
                                  ssems[b]).wait()
        return 0

    lax.fori_loop(0, NG, _group, 0)
    plsc.subcore_barrier()
    pltpu.sync_copy(shacc.at[pl.ds(s * STRIPE, STRIPE)],
                    acc_hbm.at[c, pl.ds(s * STRIPE, STRIPE)])


BLK = 2000


def _mm_body(x_ref, w_ref, xw_ref):
    xw_ref[...] = jnp.dot(x_ref[...], w_ref[...],
                          preferred_element_type=jnp.float32)


def _scale_body(xw_ref, dega_ref, degb_ref, y_ref, dinv_ref):
    deg = dega_ref[...] + degb_ref[...] + 1.0
    dinv = lax.rsqrt(deg)
    dinv_ref[...] = dinv
    y_ref[...] = xw_ref[...] * dinv


def _combine_body(acc_ref, y_ref, dinv_ref, b_ref, z_ref):
    t = acc_ref[0] + acc_ref[1] + y_ref[...]
    z_ref[...] = t * dinv_ref[...] + b_ref[...]


def kernel(edges, features, W1, b1, W2, b2):
    npad_e = EP - E
    padidx = N + jnp.arange(npad_e, dtype=jnp.int32) % (NPAD - N)
    src = jnp.concatenate([edges[0], padidx]).reshape(NW, NG, GRP, CHUNK)
    dst = jnp.concatenate([edges[1], padidx]).reshape(NW, NG, GRP, CHUNK)

    deg = _deg_kernel(dst)
    dega = deg[0].reshape(NPAD, 1)
    degb = deg[1].reshape(NPAD, 1)

    grid = N // BLK
    xw = pl.pallas_call(
        _mm_body,
        grid=(grid,),
        in_specs=[
            pl.BlockSpec((BLK, D), lambda j: (j, 0)),
            pl.BlockSpec((D, D), lambda j: (0, 0)),
        ],
        out_specs=pl.BlockSpec((BLK, D), lambda j: (j, 0)),
        out_shape=jax.ShapeDtypeStruct((N, D), jnp.float32),
    )(features, W2)

    y, dinv = pl.pallas_call(
        _scale_body,
        grid=(grid,),
        in_specs=[
            pl.BlockSpec((BLK, D), lambda j: (j, 0)),
            pl.BlockSpec((BLK, 1), lambda j: (j, 0)),
            pl.BlockSpec((BLK, 1), lambda j: (j, 0)),
        ],
        out_specs=[
            pl.BlockSpec((BLK, D), lambda j: (j, 0)),
            pl.BlockSpec((BLK, 1), lambda j: (j, 0)),
        ],
        out_shape=[
            jax.ShapeDtypeStruct((N, D), jnp.float32),
            jax.ShapeDtypeStruct((N, 1), jnp.float32),
        ],
    )(xw, dega, degb)

    y_pad = jnp.pad(y, ((0, NPAD - N), (0, 0)))
    acc = _edge_kernel(y_pad, src, dst)

    z = pl.pallas_call(
        _combine_body,
        grid=(grid,),
        in_specs=[
            pl.BlockSpec((NC, BLK, D), lambda j: (0, j, 0)),
            pl.BlockSpec((BLK, D), lambda j: (j, 0)),
            pl.BlockSpec((BLK, 1), lambda j: (j, 0)),
            pl.BlockSpec((1, D), lambda j: (0, 0)),
        ],
        out_specs=pl.BlockSpec((BLK, D), lambda j: (j, 0)),
        out_shape=jax.ShapeDtypeStruct((N, D), jnp.float32),
    )(acc, y, dinv, b2.reshape(1, D))
    return z


# speedup vs baseline: 1.0960x; 1.0960x over previous
import functools

import jax
import jax.numpy as jnp
from jax import lax
from jax.experimental import pallas as pl
from jax.experimental.pallas import tpu as pltpu
from jax.experimental.pallas import tpu_sc as plsc

N = 10000
D = 128
E = 320000
NC = 2
NS = 16
NW = NC * NS
L = 16
NPAD = 10240
STRIPE = NPAD // NS

CHUNK = 80
EPW = 10240
EP = NW * EPW
NCHUNK = EPW // CHUNK
GRP = 8
NG = NCHUNK // GRP
NBUF = 4
DCH = 128
DGRP = 5
DNG = EPW // (DCH * DGRP)

_mesh = plsc.VectorSubcoreMesh(core_axis_name="c", subcore_axis_name="s")


@functools.partial(
    pl.kernel,
    out_type=jax.ShapeDtypeStruct((NC, NPAD), jnp.float32),
    mesh=_mesh,
    scratch_types=[
        pltpu.VMEM((DGRP, DCH), jnp.int32),
        pltpu.VMEM((DCH,), jnp.float32),
        pltpu.VMEM((STRIPE,), jnp.float32),
        pltpu.VMEM_SHARED((NPAD,), jnp.float32),
    ],
)
def _deg_kernel(dst_hbm, deg_hbm, dstv, onesv, zbuf, shacc):
    c = lax.axis_index("c")
    s = lax.axis_index("s")
    w = s * NC + c

    def _zero(i, _):
        zbuf[pl.ds(i * L, L)] = jnp.zeros((L,), jnp.float32)
        return 0

    lax.fori_loop(0, STRIPE // L, _zero, 0)

    def _ones(i, _):
        onesv[pl.ds(i * L, L)] = jnp.ones((L,), jnp.float32)
        return 0

    lax.fori_loop(0, DCH // L, _ones, 0)

    pltpu.sync_copy(zbuf, shacc.at[pl.ds(s * STRIPE, STRIPE)])
    plsc.subcore_barrier()

    def _group(g, _):
        pltpu.sync_copy(dst_hbm.at[w, g], dstv)

        def _body(j, _):
            pltpu.sync_copy(onesv, shacc.at[dstv.at[j]], add=True)
            return 0

        lax.fori_loop(0, DGRP, _body, 0)
        return 0

    lax.fori_loop(0, DNG, _group, 0)
    plsc.subcore_barrier()
    pltpu.sync_copy(shacc.at[pl.ds(s * STRIPE, STRIPE)],
                    deg_hbm.at[c, pl.ds(s * STRIPE, STRIPE)])


@functools.partial(
    pl.kernel,
    out_type=jax.ShapeDtypeStruct((NC, NPAD, D), jnp.float32),
    mesh=_mesh,
    scratch_types=[
        pltpu.VMEM((GRP, CHUNK), jnp.int32),
        pltpu.VMEM((GRP, CHUNK), jnp.int32),
        [pltpu.VMEM((CHUNK, D), jnp.float32)] * NBUF,
        [pltpu.SemaphoreType.DMA] * NBUF,
        [pltpu.SemaphoreType.DMA] * NBUF,
        pltpu.VMEM_SHARED((NPAD, D), jnp.float32),
    ],
)
def _edge_kernel(y_hbm, src_hbm, dst_hbm, acc_hbm, srcv, dstv, bufs,
                 gsems, ssems, shacc):
    c = lax.axis_index("c")
    s = lax.axis_index("s")
    w = s * NC + c

    def _zero(i, _):
        r = i // (D // L)
        q = i % (D // L)
        bufs[0][r, pl.ds(q * L, L)] = jnp.zeros((L,), jnp.float32)
        return 0

    lax.fori_loop(0, CHUNK * (D // L), _zero, 0)

    def _zcopy(k, _):
        pltpu.sync_copy(bufs[0], shacc.at[pl.ds(s * STRIPE + k * CHUNK, CHUNK)])
        return 0

    lax.fori_loop(0, STRIPE // CHUNK, _zcopy, 0)
    plsc.subcore_barrier()

    def _group(g, _):
        pltpu.sync_copy(src_hbm.at[w, g], srcv)
        pltpu.sync_copy(dst_hbm.at[w, g], dstv)
        for k in range(NBUF - 1):
            pltpu.async_copy(y_hbm.at[srcv.at[k]], bufs[k], gsems[k])
        for k in range(GRP):
            b = k % NBUF
            pltpu.make_async_copy(y_hbm.at[srcv.at[k]], bufs[b],
                                  gsems[b]).wait()
            pltpu.async_copy(bufs[b], shacc.at[dstv.at[k]], ssems[b],
                             add=True)
            if k + NBUF - 1 < GRP:
                nb = (k + NBUF - 1) % NBUF
                if k - 1 >= 0:
                    pltpu.make_async_copy(bufs[nb], shacc.at[dstv.at[k - 1]],
                                          ssems[nb]).wait()
                pltpu.async_copy(y_hbm.at[srcv.at[k + NBUF - 1]], bufs[nb],
                                 gsems[nb])
        for k in range(GRP - NBUF, GRP):
            b = k % NBUF
            pltpu.make_async_copy(bufs[b], shacc.at[dstv.at[k]],
                                  ssems[b]).wait()
        return 0

    lax.fori_loop(0, NG, _group, 0)
    plsc.subcore_barrier()
    pltpu.sync_copy(shacc.at[pl.ds(s * STRIPE, STRIPE)],
                    acc_hbm.at[c, pl.ds(s * STRIPE, STRIPE)])


BLK = 2000


def _scale_mm_body(x_ref, w_ref, dega_ref, degb_ref, y_ref, dinv_ref):
    deg = dega_ref[...] + degb_ref[...] + 1.0
    dinv = lax.rsqrt(deg)
    dinv_ref[...] = dinv
    xw = jnp.dot(x_ref[...], w_ref[...], preferred_element_type=jnp.float32)
    y_ref[...] = xw * dinv


def _combine_body(acc_ref, y_ref, dinv_ref, b_ref, z_ref):
    t = acc_ref[0] + acc_ref[1] + y_ref[...]
    z_ref[...] = t * dinv_ref[...] + b_ref[...]


def kernel(edges, features, W1, b1, W2, b2):
    npad_e = EP - E
    padidx = N + jnp.arange(npad_e, dtype=jnp.int32) % (NPAD - N)
    src = jnp.concatenate([edges[0], padidx]).reshape(NW, NG, GRP, CHUNK)
    dstflat = jnp.concatenate([edges[1], padidx])
    dst = dstflat.reshape(NW, NG, GRP, CHUNK)

    deg = _deg_kernel(dstflat.reshape(NW, DNG, DGRP, DCH))
    dega = deg[0].reshape(NPAD, 1)
    degb = deg[1].reshape(NPAD, 1)

    grid = N // BLK
    y, dinv = pl.pallas_call(
        _scale_mm_body,
        grid=(grid,),
        in_specs=[
            pl.BlockSpec((BLK, D), lambda j: (j, 0)),
            pl.BlockSpec((D, D), lambda j: (0, 0)),
            pl.BlockSpec((BLK, 1), lambda j: (j, 0)),
            pl.BlockSpec((BLK, 1), lambda j: (j, 0)),
        ],
        out_specs=[
            pl.BlockSpec((BLK, D), lambda j: (j, 0)),
            pl.BlockSpec((BLK, 1), lambda j: (j, 0)),
        ],
        out_shape=[
            jax.ShapeDtypeStruct((NPAD, D), jnp.float32),
            jax.ShapeDtypeStruct((N, 1), jnp.float32),
        ],
    )(features, W2, dega, degb)

    acc = _edge_kernel(y, src, dst)

    z = pl.pallas_call(
        _combine_body,
        grid=(grid,),
        in_specs=[
            pl.BlockSpec((NC, BLK, D), lambda j: (0, j, 0)),
            pl.BlockSpec((BLK, D), lambda j: (j, 0)),
            pl.BlockSpec((BLK, 1), lambda j: (j, 0)),
            pl.BlockSpec((1, D), lambda j: (0, 0)),
        ],
        out_specs=pl.BlockSpec((BLK, D), lambda j: (j, 0)),
        out_shape=jax.ShapeDtypeStruct((N, D), jnp.float32),
    )(acc, y, dinv, b2.reshape(1, D))
    return z


# speedup vs baseline: 1.1865x; 1.0826x over previous
import functools

import jax
import jax.numpy as jnp
from jax import lax
from jax.experimental import pallas as pl
from jax.experimental.pallas import tpu as pltpu
from jax.experimental.pallas import tpu_sc as plsc

N = 10000
D = 128
E = 320000
NC = 2
NS = 16
NW = NC * NS
L = 16
NPAD = 10240
STRIPE = NPAD // NS

CHUNK = 80
EPW = 10240
EP = NW * EPW
NCHUNK = EPW // CHUNK
GRP = 16
NG = NCHUNK // GRP
NBUF = 4
DCH = 128
DGRP = 5
DNG = EPW // (DCH * DGRP)

_mesh = plsc.VectorSubcoreMesh(core_axis_name="c", subcore_axis_name="s")


@functools.partial(
    pl.kernel,
    out_type=jax.ShapeDtypeStruct((NC, NPAD), jnp.float32),
    mesh=_mesh,
    scratch_types=[
        pltpu.VMEM((DGRP, DCH), jnp.int32),
        pltpu.VMEM((DCH,), jnp.float32),
        pltpu.VMEM((STRIPE,), jnp.float32),
        pltpu.VMEM_SHARED((NPAD,), jnp.float32),
    ],
)
def _deg_kernel(dst_hbm, deg_hbm, dstv, onesv, zbuf, shacc):
    c = lax.axis_index("c")
    s = lax.axis_index("s")
    w = s * NC + c

    def _zero(i, _):
        zbuf[pl.ds(i * L, L)] = jnp.zeros((L,), jnp.float32)
        return 0

    lax.fori_loop(0, STRIPE // L, _zero, 0)

    def _ones(i, _):
        onesv[pl.ds(i * L, L)] = jnp.ones((L,), jnp.float32)
        return 0

    lax.fori_loop(0, DCH // L, _ones, 0)

    pltpu.sync_copy(zbuf, shacc.at[pl.ds(s * STRIPE, STRIPE)])
    plsc.subcore_barrier()

    def _group(g, _):
        pltpu.sync_copy(dst_hbm.at[w, g], dstv)

        def _body(j, _):
            pltpu.sync_copy(onesv, shacc.at[dstv.at[j]], add=True)
            return 0

        lax.fori_loop(0, DGRP, _body, 0)
        return 0

    lax.fori_loop(0, DNG, _group, 0)
    plsc.subcore_barrier()
    pltpu.sync_copy(shacc.at[pl.ds(s * STRIPE, STRIPE)],
                    deg_hbm.at[c, pl.ds(s * STRIPE, STRIPE)])


@functools.partial(
    pl.kernel,
    out_type=jax.ShapeDtypeStruct((NC, NPAD, D), jnp.float32),
    mesh=_mesh,
    scratch_types=[
        pltpu.VMEM((GRP, CHUNK), jnp.int32),
        pltpu.VMEM((GRP, CHUNK), jnp.int32),
        [pltpu.VMEM((CHUNK, D), jnp.float32)] * NBUF,
        [pltpu.SemaphoreType.DMA] * NBUF,
        [pltpu.SemaphoreType.DMA] * NBUF,
        pltpu.VMEM_SHARED((NPAD, D), jnp.float32),
    ],
)
def _edge_kernel(y_hbm, src_hbm, dst_hbm, acc_hbm, srcv, dstv, bufs,
                 gsems, ssems, shacc):
    c = lax.axis_index("c")
    s = lax.axis_index("s")
    w = s * NC + c

    def _zero(i, _):
        r = i // (D // L)
        q = i % (D // L)
        bufs[0][r, pl.ds(q * L, L)] = jnp.zeros((L,), jnp.float32)
        return 0

    lax.fori_loop(0, CHUNK * (D // L), _zero, 0)

    def _zcopy(k, _):
        pltpu.sync_copy(bufs[0], shacc.at[pl.ds(s * STRIPE + k * CHUNK, CHUNK)])
        return 0

    lax.fori_loop(0, STRIPE // CHUNK, _zcopy, 0)
    plsc.subcore_barrier()

    def _group(g, _):
        pltpu.sync_copy(src_hbm.at[w, g], srcv)
        pltpu.sync_copy(dst_hbm.at[w, g], dstv)
        for k in range(NBUF - 1):
            pltpu.async_copy(y_hbm.at[srcv.at[k]], bufs[k], gsems[k])
        for k in range(GRP):
            b = k % NBUF
            pltpu.make_async_copy(y_hbm.at[srcv.at[k]], bufs[b],
                                  gsems[b]).wait()
            pltpu.async_copy(bufs[b], shacc.at[dstv.at[k]], ssems[b],
                             add=True)
            if k + NBUF - 1 < GRP:
                nb = (k + NBUF - 1) % NBUF
                if k - 1 >= 0:
                    pltpu.make_async_copy(bufs[nb], shacc.at[dstv.at[k - 1]],
                                          ssems[nb]).wait()
                pltpu.async_copy(y_hbm.at[srcv.at[k + NBUF - 1]], bufs[nb],
                                 gsems[nb])
        for k in range(GRP - NBUF, GRP):
            b = k % NBUF
            pltpu.make_async_copy(bufs[b], shacc.at[dstv.at[k]],
                                  ssems[b]).wait()
        return 0

    lax.fori_loop(0, NG, _group, 0)
    plsc.subcore_barrier()
    pltpu.sync_copy(shacc.at[pl.ds(s * STRIPE, STRIPE)],
                    acc_hbm.at[c, pl.ds(s * STRIPE, STRIPE)])


BLK = 2000


def _scale_mm_body(x_ref, w_ref, dega_ref, degb_ref, y_ref, dinv_ref):
    deg = dega_ref[...] + degb_ref[...] + 1.0
    dinv = lax.rsqrt(deg)
    dinv_ref[...] = dinv
    xw = jnp.dot(x_ref[...], w_ref[...], preferred_element_type=jnp.float32)
    y_ref[...] = xw * dinv


def _combine_body(acc_ref, y_ref, dinv_ref, b_ref, z_ref):
    t = acc_ref[0] + acc_ref[1] + y_ref[...]
    z_ref[...] = t * dinv_ref[...] + b_ref[...]


def kernel(edges, features, W1, b1, W2, b2):
    npad_e = EP - E
    padidx = N + jnp.arange(npad_e, dtype=jnp.int32) % (NPAD - N)
    src = jnp.concatenate([edges[0], padidx]).reshape(NW, NG, GRP, CHUNK)
    dstflat = jnp.concatenate([edges[1], padidx])
    dst = dstflat.reshape(NW, NG, GRP, CHUNK)

    deg = _deg_kernel(dstflat.reshape(NW, DNG, DGRP, DCH))
    dega = deg[0].reshape(NPAD, 1)
    degb = deg[1].reshape(NPAD, 1)

    grid = N // BLK
    y, dinv = pl.pallas_call(
        _scale_mm_body,
        grid=(grid,),
        in_specs=[
            pl.BlockSpec((BLK, D), lambda j: (j, 0)),
            pl.BlockSpec((D, D), lambda j: (0, 0)),
            pl.BlockSpec((BLK, 1), lambda j: (j, 0)),
            pl.BlockSpec((BLK, 1), lambda j: (j, 0)),
        ],
        out_specs=[
            pl.BlockSpec((BLK, D), lambda j: (j, 0)),
            pl.BlockSpec((BLK, 1), lambda j: (j, 0)),
        ],
        out_shape=[
            jax.ShapeDtypeStruct((NPAD, D), jnp.float32),
            jax.ShapeDtypeStruct((N, 1), jnp.float32),
        ],
    )(features, W2, dega, degb)

    acc = _edge_kernel(y, src, dst)

    z = pl.pallas_call(
        _combine_body,
        grid=(grid,),
        in_specs=[
            pl.BlockSpec((NC, BLK, D), lambda j: (0, j, 0)),
            pl.BlockSpec((BLK, D), lambda j: (j, 0)),
            pl.BlockSpec((BLK, 1), lambda j: (j, 0)),
            pl.BlockSpec((1, D), lambda j: (0, 0)),
        ],
        out_specs=pl.BlockSpec((BLK, D), lambda j: (j, 0)),
        out_shape=jax.ShapeDtypeStruct((N, D), jnp.float32),
    )(acc, y, dinv, b2.reshape(1, D))
    return z


# speedup vs baseline: 1.2371x; 1.0426x over previous
import functools

import jax
import jax.numpy as jnp
from jax import lax
from jax.experimental import pallas as pl
from jax.experimental.pallas import tpu as pltpu
from jax.experimental.pallas import tpu_sc as plsc

N = 10000
D = 128
E = 320000
NC = 2
NS = 16
NW = NC * NS
L = 16
NPAD = 10240
STRIPE = NPAD // NS

CHUNK = 80
EPW = 10240
EP = NW * EPW
NCHUNK = EPW // CHUNK
GRP = 32
NG = NCHUNK // GRP
NBUF = 4
DCH = 128
DGRP = 5
DNG = EPW // (DCH * DGRP)

_mesh = plsc.VectorSubcoreMesh(core_axis_name="c", subcore_axis_name="s")


@functools.partial(
    pl.kernel,
    out_type=jax.ShapeDtypeStruct((NC, NPAD), jnp.float32),
    mesh=_mesh,
    scratch_types=[
        pltpu.VMEM((DGRP, DCH), jnp.int32),
        pltpu.VMEM((DCH,), jnp.float32),
        pltpu.VMEM((STRIPE,), jnp.float32),
        pltpu.VMEM_SHARED((NPAD,), jnp.float32),
    ],
)
def _deg_kernel(dst_hbm, deg_hbm, dstv, onesv, zbuf, shacc):
    c = lax.axis_index("c")
    s = lax.axis_index("s")
    w = s * NC + c

    def _zero(i, _):
        zbuf[pl.ds(i * L, L)] = jnp.zeros((L,), jnp.float32)
        return 0

    lax.fori_loop(0, STRIPE // L, _zero, 0)

    def _ones(i, _):
        onesv[pl.ds(i * L, L)] = jnp.ones((L,), jnp.float32)
        return 0

    lax.fori_loop(0, DCH // L, _ones, 0)

    pltpu.sync_copy(zbuf, shacc.at[pl.ds(s * STRIPE, STRIPE)])
    plsc.subcore_barrier()

    def _group(g, _):
        pltpu.sync_copy(dst_hbm.at[w, g], dstv)

        def _body(j, _):
            pltpu.sync_copy(onesv, shacc.at[dstv.at[j]], add=True)
            return 0

        lax.fori_loop(0, DGRP, _body, 0)
        return 0

    lax.fori_loop(0, DNG, _group, 0)
    plsc.subcore_barrier()
    pltpu.sync_copy(shacc.at[pl.ds(s * STRIPE, STRIPE)],
                    deg_hbm.at[c, pl.ds(s * STRIPE, STRIPE)])


@functools.partial(
    pl.kernel,
    out_type=jax.ShapeDtypeStruct((NC, NPAD, D), jnp.float32),
    mesh=_mesh,
    scratch_types=[
        pltpu.VMEM((GRP, CHUNK), jnp.int32),
        pltpu.VMEM((GRP, CHUNK), jnp.int32),
        [pltpu.VMEM((CHUNK, D), jnp.float32)] * NBUF,
        [pltpu.SemaphoreType.DMA] * NBUF,
        [pltpu.SemaphoreType.DMA] * NBUF,
        pltpu.VMEM_SHARED((NPAD, D), jnp.float32),
    ],
)
def _edge_kernel(y_hbm, src_hbm, dst_hbm, acc_hbm, srcv, dstv, bufs,
                 gsems, ssems, shacc):
    c = lax.axis_index("c")
    s = lax.axis_index("s")
    w = s * NC + c

    def _zero(i, _):
        r = i // (D // L)
        q = i % (D // L)
        bufs[0][r, pl.ds(q * L, L)] = jnp.zeros((L,), jnp.float32)
        return 0

    lax.fori_loop(0, CHUNK * (D // L), _zero, 0)

    def _zcopy(k, _):
        pltpu.sync_copy(bufs[0], shacc.at[pl.ds(s * STRIPE + k * CHUNK, CHUNK)])
        return 0

    lax.fori_loop(0, STRIPE // CHUNK, _zcopy, 0)
    plsc.subcore_barrier()

    def _group(g, _):
        pltpu.sync_copy(src_hbm.at[w, g], srcv)
        pltpu.sync_copy(dst_hbm.at[w, g], dstv)
        for k in range(NBUF - 1):
            pltpu.async_copy(y_hbm.at[srcv.at[k]], bufs[k], gsems[k])
        for k in range(GRP):
            b = k % NBUF
            pltpu.make_async_copy(y_hbm.at[srcv.at[k]], bufs[b],
                                  gsems[b]).wait()
            pltpu.async_copy(bufs[b], shacc.at[dstv.at[k]], ssems[b],
                             add=True)
            if k + NBUF - 1 < GRP:
                nb = (k + NBUF - 1) % NBUF
                if k - 1 >= 0:
                    pltpu.make_async_copy(bufs[nb], shacc.at[dstv.at[k - 1]],
                                          ssems[nb]).wait()
                pltpu.async_copy(y_hbm.at[srcv.at[k + NBUF - 1]], bufs[nb],
                                 gsems[nb])
        for k in range(GRP - NBUF, GRP):
            b = k % NBUF
            pltpu.make_async_copy(bufs[b], shacc.at[dstv.at[k]],
                                  ssems[b]).wait()
        return 0

    lax.fori_loop(0, NG, _group, 0)
    plsc.subcore_barrier()
    pltpu.sync_copy(shacc.at[pl.ds(s * STRIPE, STRIPE)],
                    acc_hbm.at[c, pl.ds(s * STRIPE, STRIPE)])


BLK = 2000


def _scale_mm_body(x_ref, w_ref, dega_ref, degb_ref, y_ref, dinv_ref):
    deg = dega_ref[...] + degb_ref[...] + 1.0
    dinv = lax.rsqrt(deg)
    dinv_ref[...] = dinv
    xw = jnp.dot(x_ref[...], w_ref[...], preferred_element_type=jnp.float32)
    y_ref[...] = xw * dinv


def _combine_body(acc_ref, y_ref, dinv_ref, b_ref, z_ref):
    t = acc_ref[0] + acc_ref[1] + y_ref[...]
    z_ref[...] = t * dinv_ref[...] + b_ref[...]


def kernel(edges, features, W1, b1, W2, b2):
    npad_e = EP - E
    padidx = N + jnp.arange(npad_e, dtype=jnp.int32) % (NPAD - N)
    src = jnp.concatenate([edges[0], padidx]).reshape(NW, NG, GRP, CHUNK)
    dstflat = jnp.concatenate([edges[1], padidx])
    dst = dstflat.reshape(NW, NG, GRP, CHUNK)

    deg = _deg_kernel(dstflat.reshape(NW, DNG, DGRP, DCH))
    dega = deg[0].reshape(NPAD, 1)
    degb = deg[1].reshape(NPAD, 1)

    grid = N // BLK
    y, dinv = pl.pallas_call(
        _scale_mm_body,
        grid=(grid,),
        in_specs=[
            pl.BlockSpec((BLK, D), lambda j: (j, 0)),
            pl.BlockSpec((D, D), lambda j: (0, 0)),
            pl.BlockSpec((BLK, 1), lambda j: (j, 0)),
            pl.BlockSpec((BLK, 1), lambda j: (j, 0)),
        ],
        out_specs=[
            pl.BlockSpec((BLK, D), lambda j: (j, 0)),
            pl.BlockSpec((BLK, 1), lambda j: (j, 0)),
        ],
        out_shape=[
            jax.ShapeDtypeStruct((NPAD, D), jnp.float32),
            jax.ShapeDtypeStruct((N, 1), jnp.float32),
        ],
    )(features, W2, dega, degb)

    acc = _edge_kernel(y, src, dst)

    z = pl.pallas_call(
        _combine_body,
        grid=(grid,),
        in_specs=[
            pl.BlockSpec((NC, BLK, D), lambda j: (0, j, 0)),
            pl.BlockSpec((BLK, D), lambda j: (j, 0)),
            pl.BlockSpec((BLK, 1), lambda j: (j, 0)),
            pl.BlockSpec((1, D), lambda j: (0, 0)),
        ],
        out_specs=pl.BlockSpec((BLK, D), lambda j: (j, 0)),
        out_shape=jax.ShapeDtypeStruct((N, D), jnp.float32),
    )(acc, y, dinv, b2.reshape(1, D))
    return z


# speedup vs baseline: 1.2559x; 1.0152x over previous
import functools

import jax
import jax.numpy as jnp
from jax import lax
from jax.experimental import pallas as pl
from jax.experimental.pallas import tpu as pltpu
from jax.experimental.pallas import tpu_sc as plsc

N = 10000
D = 128
E = 320000
NC = 2
NS = 16
NW = NC * NS
L = 16
NPAD = 10240
STRIPE = NPAD // NS

CHUNK = 64
EPW = 10240
EP = NW * EPW
NCHUNK = EPW // CHUNK
GRP = 32
NG = NCHUNK // GRP
NBUF = 5
DCH = 128
DGRP = 5
DNG = EPW // (DCH * DGRP)

_mesh = plsc.VectorSubcoreMesh(core_axis_name="c", subcore_axis_name="s")


@functools.partial(
    pl.kernel,
    out_type=jax.ShapeDtypeStruct((NC, NPAD), jnp.float32),
    mesh=_mesh,
    scratch_types=[
        pltpu.VMEM((DGRP, DCH), jnp.int32),
        pltpu.VMEM((DCH,), jnp.float32),
        pltpu.VMEM((STRIPE,), jnp.float32),
        pltpu.VMEM_SHARED((NPAD,), jnp.float32),
    ],
)
def _deg_kernel(dst_hbm, deg_hbm, dstv, onesv, zbuf, shacc):
    c = lax.axis_index("c")
    s = lax.axis_index("s")
    w = s * NC + c

    def _zero(i, _):
        zbuf[pl.ds(i * L, L)] = jnp.zeros((L,), jnp.float32)
        return 0

    lax.fori_loop(0, STRIPE // L, _zero, 0)

    def _ones(i, _):
        onesv[pl.ds(i * L, L)] = jnp.ones((L,), jnp.float32)
        return 0

    lax.fori_loop(0, DCH // L, _ones, 0)

    pltpu.sync_copy(zbuf, shacc.at[pl.ds(s * STRIPE, STRIPE)])
    plsc.subcore_barrier()

    def _group(g, _):
        pltpu.sync_copy(dst_hbm.at[w, g], dstv)

        def _body(j, _):
            pltpu.sync_copy(onesv, shacc.at[dstv.at[j]], add=True)
            return 0

        lax.fori_loop(0, DGRP, _body, 0)
        return 0

    lax.fori_loop(0, DNG, _group, 0)
    plsc.subcore_barrier()
    pltpu.sync_copy(shacc.at[pl.ds(s * STRIPE, STRIPE)],
                    deg_hbm.at[c, pl.ds(s * STRIPE, STRIPE)])


@functools.partial(
    pl.kernel,
    out_type=jax.ShapeDtypeStruct((NC, NPAD, D), jnp.float32),
    mesh=_mesh,
    scratch_types=[
        pltpu.VMEM((GRP, CHUNK), jnp.int32),
        pltpu.VMEM((GRP, CHUNK), jnp.int32),
        [pltpu.VMEM((CHUNK, D), jnp.float32)] * NBUF,
        [pltpu.SemaphoreType.DMA] * NBUF,
        [pltpu.SemaphoreType.DMA] * NBUF,
        pltpu.VMEM_SHARED((NPAD, D), jnp.float32),
    ],
)
def _edge_kernel(y_hbm, src_hbm, dst_hbm, acc_hbm, srcv, dstv, bufs,
                 gsems, ssems, shacc):
    c = lax.axis_index("c")
    s = lax.axis_index("s")
    w = s * NC + c

    def _zero(i, _):
        r = i // (D // L)
        q = i % (D // L)
        bufs[0][r, pl.ds(q * L, L)] = jnp.zeros((L,), jnp.float32)
        return 0

    lax.fori_loop(0, CHUNK * (D // L), _zero, 0)

    def _zcopy(k, _):
        pltpu.sync_copy(bufs[0], shacc.at[pl.ds(s * STRIPE + k * CHUNK, CHUNK)])
        return 0

    lax.fori_loop(0, STRIPE // CHUNK, _zcopy, 0)
    plsc.subcore_barrier()

    def _group(g, _):
        pltpu.sync_copy(src_hbm.at[w, g], srcv)
        pltpu.sync_copy(dst_hbm.at[w, g], dstv)
        for k in range(NBUF - 1):
            pltpu.async_copy(y_hbm.at[srcv.at[k]], bufs[k], gsems[k])
        for k in range(GRP):
            b = k % NBUF
            pltpu.make_async_copy(y_hbm.at[srcv.at[k]], bufs[b],
                                  gsems[b]).wait()
            pltpu.async_copy(bufs[b], shacc.at[dstv.at[k]], ssems[b],
                             add=True)
            if k + NBUF - 1 < GRP:
                nb = (k + NBUF - 1) % NBUF
                if k - 1 >= 0:
                    pltpu.make_async_copy(bufs[nb], shacc.at[dstv.at[k - 1]],
                                          ssems[nb]).wait()
                pltpu.async_copy(y_hbm.at[srcv.at[k + NBUF - 1]], bufs[nb],
                                 gsems[nb])
        for k in range(GRP - NBUF, GRP):
            b = k % NBUF
            pltpu.make_async_copy(bufs[b], shacc.at[dstv.at[k]],
                                  ssems[b]).wait()
        return 0

    lax.fori_loop(0, NG, _group, 0)
    plsc.subcore_barrier()
    pltpu.sync_copy(shacc.at[pl.ds(s * STRIPE, STRIPE)],
                    acc_hbm.at[c, pl.ds(s * STRIPE, STRIPE)])


BLK = 2000


def _scale_mm_body(x_ref, w_ref, dega_ref, degb_ref, y_ref, dinv_ref):
    deg = dega_ref[...] + degb_ref[...] + 1.0
    dinv = lax.rsqrt(deg)
    dinv_ref[...] = dinv
    xw = jnp.dot(x_ref[...], w_ref[...], preferred_element_type=jnp.float32)
    y_ref[...] = xw * dinv


def _combine_body(acc_ref, y_ref, dinv_ref, b_ref, z_ref):
    t = acc_ref[0] + acc_ref[1] + y_ref[...]
    z_ref[...] = t * dinv_ref[...] + b_ref[...]


def kernel(edges, features, W1, b1, W2, b2):
    npad_e = EP - E
    padidx = N + jnp.arange(npad_e, dtype=jnp.int32) % (NPAD - N)
    src = jnp.concatenate([edges[0], padidx]).reshape(NW, NG, GRP, CHUNK)
    dstflat = jnp.concatenate([edges[1], padidx])
    dst = dstflat.reshape(NW, NG, GRP, CHUNK)

    deg = _deg_kernel(dstflat.reshape(NW, DNG, DGRP, DCH))
    dega = deg[0].reshape(NPAD, 1)
    degb = deg[1].reshape(NPAD, 1)

    grid = N // BLK
    y, dinv = pl.pallas_call(
        _scale_mm_body,
        grid=(grid,),
        in_specs=[
            pl.BlockSpec((BLK, D), lambda j: (j, 0)),
            pl.BlockSpec((D, D), lambda j: (0, 0)),
            pl.BlockSpec((BLK, 1), lambda j: (j, 0)),
            pl.BlockSpec((BLK, 1), lambda j: (j, 0)),
        ],
        out_specs=[
            pl.BlockSpec((BLK, D), lambda j: (j, 0)),
            pl.BlockSpec((BLK, 1), lambda j: (j, 0)),
        ],
        out_shape=[
            jax.ShapeDtypeStruct((NPAD, D), jnp.float32),
            jax.ShapeDtypeStruct((N, 1), jnp.float32),
        ],
    )(features, W2, dega, degb)

    acc = _edge_kernel(y, src, dst)

    z = pl.pallas_call(
        _combine_body,
        grid=(grid,),
        in_specs=[
            pl.BlockSpec((NC, BLK, D), lambda j: (0, j, 0)),
            pl.BlockSpec((BLK, D), lambda j: (j, 0)),
            pl.BlockSpec((BLK, 1), lambda j: (j, 0)),
            pl.BlockSpec((1, D), lambda j: (0, 0)),
        ],
        out_specs=pl.BlockSpec((BLK, D), lambda j: (j, 0)),
        out_shape=jax.ShapeDtypeStruct((N, D), jnp.float32),
    )(acc, y, dinv, b2.reshape(1, D))
    return z


# speedup vs baseline: 1.2848x; 1.0230x over previous
import functools

import jax
import jax.numpy as jnp
from jax import lax
from jax.experimental import pallas as pl
from jax.experimental.pallas import tpu as pltpu
from jax.experimental.pallas import tpu_sc as plsc

N = 10000
D = 128
E = 320000
NC = 2
NS = 16
NW = NC * NS
L = 16
NPAD = 10240
STRIPE = NPAD // NS

CHUNK = 64
EPW = 10240
EP = NW * EPW
NCHUNK = EPW // CHUNK
GRP = 32
NG = NCHUNK // GRP
NBUF = 5
DCH = 128
DGRP = 5
DNG = EPW // (DCH * DGRP)

_mesh = plsc.VectorSubcoreMesh(core_axis_name="c", subcore_axis_name="s")


@functools.partial(
    pl.kernel,
    out_type=jax.ShapeDtypeStruct((NC, NPAD), jnp.float32),
    mesh=_mesh,
    scratch_types=[
        pltpu.VMEM((DGRP, DCH), jnp.int32),
        pltpu.VMEM((DCH,), jnp.float32),
        pltpu.VMEM((STRIPE,), jnp.float32),
        pltpu.VMEM_SHARED((NPAD,), jnp.float32),
        pltpu.SemaphoreType.DMA,
    ],
)
def _deg_kernel(dst_hbm, deg_hbm, dstv, onesv, zbuf, shacc, dsem):
    c = lax.axis_index("c")
    s = lax.axis_index("s")
    w = s * NC + c

    def _zero(i, _):
        zbuf[pl.ds(i * L, L)] = jnp.zeros((L,), jnp.float32)
        return 0

    lax.fori_loop(0, STRIPE // L, _zero, 0)

    def _ones(i, _):
        onesv[pl.ds(i * L, L)] = jnp.ones((L,), jnp.float32)
        return 0

    lax.fori_loop(0, DCH // L, _ones, 0)

    pltpu.sync_copy(zbuf, shacc.at[pl.ds(s * STRIPE, STRIPE)])
    plsc.subcore_barrier()

    def _group(g, _):
        pltpu.sync_copy(dst_hbm.at[w, g], dstv)
        for j in range(DGRP):
            pltpu.async_copy(onesv, shacc.at[dstv.at[j]], dsem, add=True)
        for j in range(DGRP):
            pltpu.make_async_copy(onesv, shacc.at[dstv.at[j]], dsem).wait()
        return 0

    lax.fori_loop(0, DNG, _group, 0)
    plsc.subcore_barrier()
    pltpu.sync_copy(shacc.at[pl.ds(s * STRIPE, STRIPE)],
                    deg_hbm.at[c, pl.ds(s * STRIPE, STRIPE)])


@functools.partial(
    pl.kernel,
    out_type=jax.ShapeDtypeStruct((NC, NPAD, D), jnp.float32),
    mesh=_mesh,
    scratch_types=[
        pltpu.VMEM((GRP, CHUNK), jnp.int32),
        pltpu.VMEM((GRP, CHUNK), jnp.int32),
        [pltpu.VMEM((CHUNK, D), jnp.float32)] * NBUF,
        [pltpu.SemaphoreType.DMA] * NBUF,
        [pltpu.SemaphoreType.DMA] * NBUF,
        pltpu.VMEM_SHARED((NPAD, D), jnp.float32),
    ],
)
def _edge_kernel(y_hbm, src_hbm, dst_hbm, acc_hbm, srcv, dstv, bufs,
                 gsems, ssems, shacc):
    c = lax.axis_index("c")
    s = lax.axis_index("s")
    w = s * NC + c

    def _zero(i, _):
        r = i // (D // L)
        q = i % (D // L)
        bufs[0][r, pl.ds(q * L, L)] = jnp.zeros((L,), jnp.float32)
        return 0

    lax.fori_loop(0, CHUNK * (D // L), _zero, 0)

    def _zcopy(k, _):
        pltpu.sync_copy(bufs[0], shacc.at[pl.ds(s * STRIPE + k * CHUNK, CHUNK)])
        return 0

    lax.fori_loop(0, STRIPE // CHUNK, _zcopy, 0)
    plsc.subcore_barrier()

    def _group(g, _):
        pltpu.sync_copy(src_hbm.at[w, g], srcv)
        pltpu.sync_copy(dst_hbm.at[w, g], dstv)
        for k in range(NBUF - 1):
            pltpu.async_copy(y_hbm.at[srcv.at[k]], bufs[k], gsems[k])
        for k in range(GRP):
            b = k % NBUF
            pltpu.make_async_copy(y_hbm.at[srcv.at[k]], bufs[b],
                                  gsems[b]).wait()
            pltpu.async_copy(bufs[b], shacc.at[dstv.at[k]], ssems[b],
                             add=True)
            if k + NBUF - 1 < GRP:
                nb = (k + NBUF - 1) % NBUF
                if k - 1 >= 0:
                    pltpu.make_async_copy(bufs[nb], shacc.at[dstv.at[k - 1]],
                                          ssems[nb]).wait()
                pltpu.async_copy(y_hbm.at[srcv.at[k + NBUF - 1]], bufs[nb],
                                 gsems[nb])
        for k in range(GRP - NBUF, GRP):
            b = k % NBUF
            pltpu.make_async_copy(bufs[b], shacc.at[dstv.at[k]],
                                  ssems[b]).wait()
        return 0

    lax.fori_loop(0, NG, _group, 0)
    plsc.subcore_barrier()
    pltpu.sync_copy(shacc.at[pl.ds(s * STRIPE, STRIPE)],
                    acc_hbm.at[c, pl.ds(s * STRIPE, STRIPE)])


BLK = 2000


def _scale_mm_body(x_ref, w_ref, dega_ref, degb_ref, y_ref, dinv_ref):
    deg = dega_ref[...] + degb_ref[...] + 1.0
    dinv = lax.rsqrt(deg)
    dinv_ref[...] = dinv
    xw = jnp.dot(x_ref[...], w_ref[...], preferred_element_type=jnp.float32)
    y_ref[...] = xw * dinv


def _combine_body(acc_ref, y_ref, dinv_ref, b_ref, z_ref):
    t = acc_ref[0] + acc_ref[1] + y_ref[...]
    z_ref[...] = t * dinv_ref[...] + b_ref[...]


def kernel(edges, features, W1, b1, W2, b2):
    npad_e = EP - E
    padidx = N + jnp.arange(npad_e, dtype=jnp.int32) % (NPAD - N)
    src = jnp.concatenate([edges[0], padidx]).reshape(NW, NG, GRP, CHUNK)
    dstflat = jnp.concatenate([edges[1], padidx])
    dst = dstflat.reshape(NW, NG, GRP, CHUNK)

    deg = _deg_kernel(dstflat.reshape(NW, DNG, DGRP, DCH))
    dega = deg[0].reshape(NPAD, 1)
    degb = deg[1].reshape(NPAD, 1)

    grid = N // BLK
    y, dinv = pl.pallas_call(
        _scale_mm_body,
        grid=(grid,),
        in_specs=[
            pl.BlockSpec((BLK, D), lambda j: (j, 0)),
            pl.BlockSpec((D, D), lambda j: (0, 0)),
            pl.BlockSpec((BLK, 1), lambda j: (j, 0)),
            pl.BlockSpec((BLK, 1), lambda j: (j, 0)),
        ],
        out_specs=[
            pl.BlockSpec((BLK, D), lambda j: (j, 0)),
            pl.BlockSpec((BLK, 1), lambda j: (j, 0)),
        ],
        out_shape=[
            jax.ShapeDtypeStruct((NPAD, D), jnp.float32),
            jax.ShapeDtypeStruct((N, 1), jnp.float32),
        ],
    )(features, W2, dega, degb)

    acc = _edge_kernel(y, src, dst)

    z = pl.pallas_call(
        _combine_body,
        grid=(grid,),
        in_specs=[
            pl.BlockSpec((NC, BLK, D), lambda j: (0, j, 0)),
            pl.BlockSpec((BLK, D), lambda j: (j, 0)),
            pl.BlockSpec((BLK, 1), lambda j: (j, 0)),
            pl.BlockSpec((1, D), lambda j: (0, 0)),
        ],
        out_specs=pl.BlockSpec((BLK, D), lambda j: (j, 0)),
        out_shape=jax.ShapeDtypeStruct((N, D), jnp.float32),
    )(acc, y, dinv, b2.reshape(1, D))
    return z
